# Initial kernel scaffold; baseline (speedup 1.0000x reference)
import jax
import jax.numpy as jnp
from jax.experimental import pallas as pl


def kernel(x, edge_index, weight, bias):
    raise NotImplementedError("write your pallas kernel here")


# speedup vs baseline: 12.2135x; 12.2135x over previous
import functools

import jax
import jax.numpy as jnp
from jax import lax
from jax.experimental import pallas as pl
from jax.experimental.pallas import tpu as pltpu
from jax.experimental.pallas import tpu_sc as plsc

NC, NS, L = 2, 16, 16
NW = NC * NS
CW = 128


def _sc_mesh():
    return plsc.VectorSubcoreMesh(
        core_axis_name="c", subcore_axis_name="s", num_cores=NC, num_subcores=NS
    )


@functools.partial(jax.jit, static_argnums=(1, 2))
def _hist_sc(row2d, npad, chunks):
    npt = npad // NS

    def body(row_hbm, out_hbm, idx_v, ones_v, zeros_v, hist_sh):
        c = lax.axis_index("c")
        s = lax.axis_index("s")
        wid = s * NC + c
        pltpu.sync_copy(row_hbm.at[pl.ds(wid * chunks, chunks)], idx_v)
        z16 = jnp.zeros((L,), jnp.float32)
        o16 = jnp.ones((L,), jnp.float32)
        for k in range(CW // L):
            ones_v[pl.ds(k * L, L)] = o16

        def zloop(i, carry):
            zeros_v[pl.ds(i * L, L)] = z16
            return carry

        lax.fori_loop(0, npt // L, zloop, 0)
        pltpu.sync_copy(zeros_v, hist_sh.at[pl.ds(s * npt, npt)])
        plsc.subcore_barrier()

        def chunk(j, carry):
            pltpu.sync_copy(ones_v, hist_sh.at[idx_v.at[j]], add=True)
            return carry

        lax.fori_loop(0, chunks, chunk, 0)
        plsc.subcore_barrier()
        pltpu.sync_copy(hist_sh.at[pl.ds(s * npt, npt)],
                        out_hbm.at[c, pl.ds(s * npt, npt)])

    return pl.kernel(
        body,
        out_type=jax.ShapeDtypeStruct((NC, npad), jnp.float32),
        mesh=_sc_mesh(),
        scratch_types=[
            pltpu.VMEM((chunks, CW), jnp.int32),
            pltpu.VMEM((CW,), jnp.float32),
            pltpu.VMEM((npt,), jnp.float32),
            pltpu.VMEM_SHARED((npad,), jnp.float32),
        ],
    )(row2d)


@functools.partial(jax.jit, static_argnums=(3, 4))
def _aggregate_sc(xs, row2d, col2d, npad, chunks):
    npt = npad // NS
    ZR = 64

    def body(xs_hbm, row_hbm, col_hbm, out_hbm, rowi_v, coli_v, rows_v,
             zero_v, acc_sh):
        c = lax.axis_index("c")
        s = lax.axis_index("s")
        wid = s * NC + c
        pltpu.sync_copy(row_hbm.at[pl.ds(wid * chunks, chunks)], rowi_v)
        pltpu.sync_copy(col_hbm.at[pl.ds(wid * chunks, chunks)], coli_v)
        z16 = jnp.zeros((L,), jnp.float32)

        def zloop(i, carry):
            for k in range(128 // L):
                zero_v[i, pl.ds(k * L, L)] = z16
            return carry

        lax.fori_loop(0, ZR, zloop, 0)
        for t in range(npt // ZR):
            pltpu.sync_copy(zero_v, acc_sh.at[pl.ds(s * npt + t * ZR, ZR)])
        plsc.subcore_barrier()

        def chunk(j, carry):
            pltpu.sync_copy(xs_hbm.at[rowi_v.at[j]], rows_v)
            pltpu.sync_copy(rows_v, acc_sh.at[coli_v.at[j]], add=True)
            return carry

        lax.fori_loop(0, chunks, chunk, 0)
        plsc.subcore_barrier()
        pltpu.sync_copy(acc_sh.at[pl.ds(s * npt, npt)],
                        out_hbm.at[c, pl.ds(s * npt, npt)])

    return pl.kernel(
        body,
        out_type=jax.ShapeDtypeStruct((NC, npad, 128), jnp.float32),
        mesh=_sc_mesh(),
        scratch_types=[
            pltpu.VMEM((chunks, CW), jnp.int32),
            pltpu.VMEM((chunks, CW), jnp.int32),
            pltpu.VMEM((CW, 128), jnp.float32),
            pltpu.VMEM((ZR, 128), jnp.float32),
            pltpu.VMEM_SHARED((npad, 128), jnp.float32),
        ],
    )(xs, row2d, col2d)


def _dinv_tc(hists):
    def body(h_ref, o_ref):
        deg = jnp.sum(h_ref[...], axis=0, keepdims=True) + 1.0
        o_ref[...] = lax.rsqrt(deg)

    return pl.pallas_call(
        body,
        out_shape=jax.ShapeDtypeStruct((1, hists.shape[1]), jnp.float32),
    )(hists)


def _scale_rows_tc(dinv_col, xpad):
    npad = xpad.shape[0]
    BR = 256

    def body(d_ref, x_ref, o_ref):
        o_ref[...] = d_ref[...] * x_ref[...]

    return pl.pallas_call(
        body,
        grid=(npad // BR,),
        in_specs=[
            pl.BlockSpec((BR, 1), lambda i: (i, 0)),
            pl.BlockSpec((BR, 128), lambda i: (i, 0)),
        ],
        out_specs=pl.BlockSpec((BR, 128), lambda i: (i, 0)),
        out_shape=jax.ShapeDtypeStruct((npad, 128), jnp.float32),
    )(dinv_col, xpad)


def _combine_tc(aparts, xs, dinv_col, w2d, b2d):
    npad = xs.shape[0]
    BR = 256

    def body(a_ref, xs_ref, d_ref, w_ref, b_ref, o_ref):
        acc = a_ref[0] + a_ref[1] + xs_ref[...]
        o_ref[...] = w_ref[...] * (d_ref[...] * acc) + b_ref[...]

    return pl.pallas_call(
        body,
        grid=(npad // BR,),
        in_specs=[
            pl.BlockSpec((NC, BR, 128), lambda i: (0, i, 0)),
            pl.BlockSpec((BR, 128), lambda i: (i, 0)),
            pl.BlockSpec((BR, 1), lambda i: (i, 0)),
            pl.BlockSpec((1, 128), lambda i: (0, 0)),
            pl.BlockSpec((1, 128), lambda i: (0, 0)),
        ],
        out_specs=pl.BlockSpec((BR, 128), lambda i: (i, 0)),
        out_shape=jax.ShapeDtypeStruct((npad, 128), jnp.float32),
    )(aparts, xs, dinv_col, w2d, b2d)


def kernel(x, edge_index, weight, bias):
    n, d = x.shape
    e = edge_index.shape[1]
    assert d == 128
    npad = ((n + 1 + 1023) // 1024) * 1024
    chunks = ((-(-e // (NW * CW)) + 7) // 8) * 8
    epad = NW * chunks * CW

    row = edge_index[0]
    col = edge_index[1]
    padv = jnp.full((epad - e,), n, dtype=jnp.int32)
    row2d = jnp.concatenate([row, padv]).reshape(NW * chunks, CW)
    col2d = jnp.concatenate([col, padv]).reshape(NW * chunks, CW)
    xpad = jnp.pad(x, ((0, npad - n), (0, 0)))

    hists = _hist_sc(row2d, npad, chunks)
    dinv_row = _dinv_tc(hists)
    dinv_col = dinv_row.reshape(npad, 1)
    xs = _scale_rows_tc(dinv_col, xpad)
    aparts = _aggregate_sc(xs, row2d, col2d, npad, chunks)
    w2d = weight.reshape(1, 128).astype(jnp.float32)
    b2d = bias.reshape(1, 128).astype(jnp.float32)
    out_full = _combine_tc(aparts, xs, dinv_col, w2d, b2d)
    return out_full[:n]
